# baseline (speedup 1.0000x reference)
import functools

import jax
import jax.numpy as jnp
from jax import lax
from jax.experimental import pallas as pl
from jax.experimental.pallas import tpu as pltpu
from jax.experimental.pallas import tpu_sc as plsc

N_PAD = 10240
RT = 256
LANES = 16


def _make_pre_kernel(E_PAD, DIN):
    EPT = E_PAD // 16
    NB = EPT // 128
    STR = N_PAD // 16
    GB = 64
    NGB = STR // GB
    mesh = plsc.VectorSubcoreMesh(core_axis_name="c", subcore_axis_name="s")

    def body(src2_hbm, dst2_hbm, perm_hbm, feat_hbm, zeros1_hbm,
             degs_hbm, degd_hbm, xperm_hbm,
             sidx_v, didx_v, ones_v, pidx_v, gbuf_v, acc_s, acc_d, sem):
        cid = lax.axis_index("c")
        tid = lax.axis_index("s")

        @pl.when(cid == 0)
        def _():
            ones16 = jnp.ones((LANES,), jnp.float32)
            for k in range(128 // LANES):
                ones_v[pl.ds(k * 16, 16)] = ones16
            pltpu.sync_copy(zeros1_hbm, acc_s.at[pl.ds(tid * STR, STR)])
            pltpu.sync_copy(zeros1_hbm, acc_d.at[pl.ds(tid * STR, STR)])
            pltpu.sync_copy(src2_hbm.at[pl.ds(tid * NB, NB)], sidx_v)
            pltpu.sync_copy(dst2_hbm.at[pl.ds(tid * NB, NB)], didx_v)
            plsc.subcore_barrier()

            def acc_body(j, _):
                pltpu.sync_copy(ones_v, acc_s.at[sidx_v.at[j]], add=True)
                pltpu.sync_copy(ones_v, acc_d.at[didx_v.at[j]], add=True)
                return 0
            lax.fori_loop(0, NB, acc_body, 0)
            plsc.subcore_barrier()
            pltpu.sync_copy(acc_s.at[pl.ds(tid * STR, STR)],
                            degs_hbm.at[pl.ds(tid * STR, STR)])
            pltpu.sync_copy(acc_d.at[pl.ds(tid * STR, STR)],
                            degd_hbm.at[pl.ds(tid * STR, STR)])

        @pl.when(cid == 1)
        def _():
            pltpu.sync_copy(perm_hbm.at[tid], pidx_v)
            for b in range(NGB):
                pltpu.async_copy(
                    feat_hbm.at[pidx_v.at[pl.ds(b * GB, GB)]], gbuf_v,
                    sem).wait()
                pltpu.sync_copy(
                    gbuf_v, xperm_hbm.at[pl.ds(tid * STR + b * GB, GB)])

    return pl.kernel(
        body,
        out_type=(jax.ShapeDtypeStruct((N_PAD,), jnp.float32),
                  jax.ShapeDtypeStruct((N_PAD,), jnp.float32),
                  jax.ShapeDtypeStruct((N_PAD, DIN), jnp.float32)),
        mesh=mesh,
        scratch_types=[pltpu.VMEM((NB, 128), jnp.int32),
                       pltpu.VMEM((NB, 128), jnp.int32),
                       pltpu.VMEM((128,), jnp.float32),
                       pltpu.VMEM((STR,), jnp.int32),
                       pltpu.VMEM((GB, DIN), jnp.float32),
                       pltpu.VMEM_SHARED((N_PAD,), jnp.float32),
                       pltpu.VMEM_SHARED((N_PAD,), jnp.float32),
                       pltpu.SemaphoreType.DMA],
    )


def _make_agg_kernel(C, E_PAD):
    EPT = E_PAD // 16
    EB = 80
    NB = EPT // EB
    QNB = NB // 4
    STR = N_PAD // 16
    CPC = C // 2
    mesh = plsc.VectorSubcoreMesh(core_axis_name="c", subcore_axis_name="s")

    def body(*refs):
        vals = refs[:C]
        src2_hbm, dst2_hbm, zeros_hbm = refs[C:C + 3]
        outs = refs[C + 3:2 * C + 3]
        scr = refs[2 * C + 3:]
        sidx_v, didx_v = scr[0], scr[1]
        gb = scr[2:6]
        acc_sh = scr[6]
        gsem = scr[7:11]
        ssem = scr[11:15]
        cid = lax.axis_index("c")
        tid = lax.axis_index("s")

        for ci in range(CPC):
            for z in range(STR // 128):
                pltpu.sync_copy(zeros_hbm,
                                acc_sh.at[pl.ds(tid * STR + z * 128, 128)])
            plsc.subcore_barrier()

            for core in range(2):
                c = core * CPC + ci

                @pl.when(cid == core)
                def _(c=c):
                    for q in range(4):
                        base = tid * NB + q * QNB
                        pltpu.sync_copy(src2_hbm.at[pl.ds(base, QNB)],
                                        sidx_v)
                        pltpu.sync_copy(dst2_hbm.at[pl.ds(base, QNB)],
                                        didx_v)
                        pltpu.async_copy(vals[c].at[sidx_v.at[0]],
                                         gb[0], gsem[0])
                        pltpu.async_copy(vals[c].at[sidx_v.at[1]],
                                         gb[1], gsem[1])

                        def eb(k, _):
                            for i in range(4):
                                b = 4 * k + i
                                nxt = (i + 2) % 4
                                pltpu.make_async_copy(
                                    vals[c].at[sidx_v.at[b]], gb[i],
                                    gsem[i]).wait()
                                pltpu.async_copy(gb[i],
                                                 acc_sh.at[didx_v.at[b]],
                                                 ssem[i], add=True)

                                @pl.when((b >= 2) & (b + 2 < QNB))
                                def _(nxt=nxt, b=b):
                                    pltpu.make_async_copy(
                                        gb[nxt],
                                        acc_sh.at[didx_v.at[b]],
                                        ssem[nxt]).wait()

                                @pl.when(b + 2 < QNB)
                                def _(nxt=nxt, b=b):
                                    pltpu.async_copy(
                                        vals[c].at[sidx_v.at[b + 2]],
                                        gb[nxt], gsem[nxt])
                            return 0
                        lax.fori_loop(0, QNB // 4, eb, 0)
                        for i in range(4):
                            pltpu.make_async_copy(
                                gb[i], acc_sh.at[didx_v.at[0]],
                                ssem[i]).wait()
            plsc.subcore_barrier()

            for core in range(2):
                c = core * CPC + ci

                @pl.when(cid == core)
                def _(c=c):
                    for z in range(STR // 128):
                        pltpu.sync_copy(
                            acc_sh.at[pl.ds(tid * STR + z * 128, 128)],
                            outs[c].at[pl.ds(tid * STR + z * 128, 128)])
            plsc.subcore_barrier()

    return pl.kernel(
        body,
        out_type=tuple(jax.ShapeDtypeStruct((N_PAD, 128), jnp.float32)
                       for _ in range(C)),
        mesh=mesh,
        scratch_types=[pltpu.VMEM((QNB, EB), jnp.int32),
                       pltpu.VMEM((QNB, EB), jnp.int32)] +
                      [pltpu.VMEM((EB, 128), jnp.float32)] * 4 +
                      [pltpu.VMEM_SHARED((N_PAD, 128), jnp.float32)] +
                      [pltpu.SemaphoreType.DMA] * 8,
    )


def _prep_body(feat_ref, xperm_ref, degs_ref, o0, o1, o2, o3):
    ns = lax.rsqrt(jnp.maximum(degs_ref[...], 1.0))
    xs = feat_ref[...] * ns
    xn = xperm_ref[...] * ns
    o0[...] = xs[:, :128]
    o1[...] = xs[:, 128:]
    o2[...] = xn[:, :128]
    o3[...] = xn[:, 128:]


def _l1_body(a0, a1_, a2, a3, W1_ref, b1_ref, al_ref, degs_ref, degd_ref,
             *h_refs):
    ns = lax.rsqrt(jnp.maximum(degs_ref[...], 1.0))
    nd = lax.rsqrt(jnp.maximum(degd_ref[...], 1.0))
    W1 = W1_ref[...]
    b1 = b1_ref[...]
    al = al_ref[...]
    aggP = jnp.concatenate([a0[...], a1_[...]], axis=1)
    aggN = jnp.concatenate([a2[...], a3[...]], axis=1)
    yp = jnp.dot(aggP, W1, preferred_element_type=jnp.float32) * nd + b1
    yn = jnp.dot(aggN, W1, preferred_element_type=jnp.float32) * nd + b1
    hp = jnp.where(yp >= 0, yp, al * yp) * ns
    hn = jnp.where(yn >= 0, yn, al * yn) * ns
    for k in range(4):
        h_refs[k][...] = hp[:, k * 128:(k + 1) * 128]
        h_refs[4 + k][...] = hn[:, k * 128:(k + 1) * 128]


def _l2_body(g0, g1, g2, g3, g4, g5, g6, g7, W2_ref, b2_ref, degd_ref,
             pos_ref, neg_ref, cs_ref, *, n_real):
    r = pl.program_id(0)
    nd = lax.rsqrt(jnp.maximum(degd_ref[...], 1.0))
    W2 = W2_ref[...]
    b2 = b2_ref[...]
    aggP = jnp.concatenate([g0[...], g1[...], g2[...], g3[...]], axis=1) * nd
    aggN = jnp.concatenate([g4[...], g5[...], g6[...], g7[...]], axis=1) * nd
    pos = jnp.dot(aggP, W2, preferred_element_type=jnp.float32) + b2
    neg = jnp.dot(aggN, W2, preferred_element_type=jnp.float32) + b2
    pos_ref[...] = pos
    neg_ref[...] = neg
    rows = r * RT + lax.broadcasted_iota(jnp.int32, (RT, 1), 0)
    posm = jnp.where(rows < n_real, pos, 0.0)
    cs = jnp.sum(posm, axis=0, keepdims=True)

    @pl.when(r == 0)
    def _():
        cs_ref[...] = cs

    @pl.when(r > 0)
    def _():
        cs_ref[...] += cs


def _softplus(x):
    return jnp.maximum(x, 0.0) + jnp.log(1.0 + jnp.exp(-jnp.abs(x)))


def _loss_body(cs_ref, Wd_ref, pos_ref, neg_ref, out_ref, ws_v, *, n_real):
    r = pl.program_id(0)

    @pl.when(r == 0)
    def _():
        s = jax.nn.sigmoid(cs_ref[...] * (1.0 / n_real))
        ws_v[...] = lax.dot_general(Wd_ref[...], s,
                                    (((1,), (1,)), ((), ())),
                                    preferred_element_type=jnp.float32)
        out_ref[...] = jnp.zeros((1, 1), jnp.float32)

    @pl.when(r > 0)
    def _():
        rows = (r - 1) * RT + lax.broadcasted_iota(jnp.int32, (RT, 1), 0)
        mask = rows < n_real
        ps = jnp.dot(pos_ref[...], ws_v[...],
                     preferred_element_type=jnp.float32)
        nsc = jnp.dot(neg_ref[...], ws_v[...],
                      preferred_element_type=jnp.float32)
        term = _softplus(-ps) + _softplus(nsc)
        acc = jnp.sum(jnp.where(mask, term, 0.0)) * (1.0 / n_real)
        out_ref[...] += acc.reshape(1, 1)


def kernel(feat, edge_index, W1, b1, a1, W2, b2, Wd):
    N, DIN = feat.shape
    DH = W1.shape[1]
    E = edge_index.shape[1]
    E_PAD = -(-E // 40960) * 40960
    GRID = N_PAD // RT
    f32 = jnp.float32

    perm = jax.random.permutation(jax.random.key(42), N).astype(jnp.int32)
    src = edge_index[0]
    dst = edge_index[1]
    padi = jnp.full((E_PAD - E,), N, jnp.int32)
    src_p = jnp.concatenate([src, padi])
    dst_p = jnp.concatenate([dst, padi])
    src2 = src_p.reshape(E_PAD // 128, 128)
    dst2 = dst_p.reshape(E_PAD // 128, 128)
    src2a = src_p.reshape(E_PAD // 80, 80)
    dst2a = dst_p.reshape(E_PAD // 80, 80)
    perm_p = jnp.concatenate([perm, jnp.zeros((N_PAD - N,), jnp.int32)])
    perm2 = perm_p.reshape(16, N_PAD // 16)
    featp = jnp.zeros((N_PAD, DIN), f32).at[:N].set(feat)
    zeros128 = jnp.zeros((128, 128), f32)
    b1r = b1.reshape(1, DH)
    a1r = a1.reshape(1, DH)
    b2r = b2.reshape(1, DH)

    zeros1 = jnp.zeros((N_PAD // 16,), f32)
    degs, degd, xperm = _make_pre_kernel(E_PAD, DIN)(
        src2, dst2, perm2, featp, zeros1)
    degs2 = degs.reshape(N_PAD, 1)
    degd2 = degd.reshape(N_PAD, 1)

    row = lambda r: (r, 0)
    fixed = lambda r: (0, 0)
    chunk_spec = pl.BlockSpec((RT, 128), row)
    b1_chunks = pl.pallas_call(
        _prep_body,
        grid=(GRID,),
        in_specs=[pl.BlockSpec((RT, DIN), row),
                  pl.BlockSpec((RT, DIN), row),
                  pl.BlockSpec((RT, 1), row)],
        out_specs=[chunk_spec] * 4,
        out_shape=[jax.ShapeDtypeStruct((N_PAD, 128), f32)] * 4,
    )(featp, xperm, degs2)

    agg1 = _make_agg_kernel(4, E_PAD)(*b1_chunks, src2a, dst2a, zeros128)

    h_chunks = pl.pallas_call(
        _l1_body,
        grid=(GRID,),
        in_specs=[chunk_spec] * 4 + [
            pl.BlockSpec((DIN, DH), fixed),
            pl.BlockSpec((1, DH), fixed),
            pl.BlockSpec((1, DH), fixed),
            pl.BlockSpec((RT, 1), row),
            pl.BlockSpec((RT, 1), row)],
        out_specs=[chunk_spec] * 8,
        out_shape=[jax.ShapeDtypeStruct((N_PAD, 128), f32)] * 8,
    )(*agg1, W1, b1r, a1r, degs2, degd2)

    agg2 = _make_agg_kernel(8, E_PAD)(*h_chunks, src2a, dst2a, zeros128)

    posA, negA, cs = pl.pallas_call(
        functools.partial(_l2_body, n_real=N),
        grid=(GRID,),
        in_specs=[chunk_spec] * 8 + [
            pl.BlockSpec((DH, DH), fixed),
            pl.BlockSpec((1, DH), fixed),
            pl.BlockSpec((RT, 1), row)],
        out_specs=[pl.BlockSpec((RT, DH), row),
                   pl.BlockSpec((RT, DH), row),
                   pl.BlockSpec((1, DH), fixed)],
        out_shape=[jax.ShapeDtypeStruct((N_PAD, DH), f32),
                   jax.ShapeDtypeStruct((N_PAD, DH), f32),
                   jax.ShapeDtypeStruct((1, DH), f32)],
    )(*agg2, W2, b2r, degd2)

    loss = pl.pallas_call(
        functools.partial(_loss_body, n_real=N),
        grid=(GRID + 1,),
        in_specs=[pl.BlockSpec((1, DH), fixed),
                  pl.BlockSpec((DH, DH), fixed),
                  pl.BlockSpec((RT, DH), lambda r: (jnp.maximum(r - 1, 0), 0)),
                  pl.BlockSpec((RT, DH), lambda r: (jnp.maximum(r - 1, 0), 0))],
        out_specs=pl.BlockSpec((1, 1), fixed),
        out_shape=jax.ShapeDtypeStruct((1, 1), f32),
        scratch_shapes=[pltpu.VMEM((DH, 1), f32)],
    )(cs, Wd, posA, negA)

    return loss[0, 0]


# speedup vs baseline: 1.1146x; 1.1146x over previous
import functools

import jax
import jax.numpy as jnp
from jax import lax
from jax.experimental import pallas as pl
from jax.experimental.pallas import tpu as pltpu
from jax.experimental.pallas import tpu_sc as plsc

N_PAD = 10240
RT = 256
LANES = 16


def _make_pre_kernel(E_PAD, DIN):
    EPT = E_PAD // 16
    NB = EPT // 128
    STR = N_PAD // 16
    GB = 64
    NGB = STR // GB
    mesh = plsc.VectorSubcoreMesh(core_axis_name="c", subcore_axis_name="s")

    def body(src2_hbm, dst2_hbm, perm_hbm, feat_hbm, zeros1_hbm,
             degs_hbm, degd_hbm, xperm_hbm,
             sidx_v, didx_v, ones_v, pidx_v, gbuf_v, acc_s, acc_d, sem):
        cid = lax.axis_index("c")
        tid = lax.axis_index("s")

        @pl.when(cid == 0)
        def _():
            ones16 = jnp.ones((LANES,), jnp.float32)
            for k in range(128 // LANES):
                ones_v[pl.ds(k * 16, 16)] = ones16
            pltpu.sync_copy(zeros1_hbm, acc_s.at[pl.ds(tid * STR, STR)])
            pltpu.sync_copy(zeros1_hbm, acc_d.at[pl.ds(tid * STR, STR)])
            pltpu.sync_copy(src2_hbm.at[pl.ds(tid * NB, NB)], sidx_v)
            pltpu.sync_copy(dst2_hbm.at[pl.ds(tid * NB, NB)], didx_v)
            plsc.subcore_barrier()

            def acc_body(j, _):
                pltpu.sync_copy(ones_v, acc_s.at[sidx_v.at[j]], add=True)
                pltpu.sync_copy(ones_v, acc_d.at[didx_v.at[j]], add=True)
                return 0
            lax.fori_loop(0, NB, acc_body, 0)
            plsc.subcore_barrier()
            pltpu.sync_copy(acc_s.at[pl.ds(tid * STR, STR)],
                            degs_hbm.at[pl.ds(tid * STR, STR)])
            pltpu.sync_copy(acc_d.at[pl.ds(tid * STR, STR)],
                            degd_hbm.at[pl.ds(tid * STR, STR)])

        @pl.when(cid == 1)
        def _():
            pltpu.sync_copy(perm_hbm.at[tid], pidx_v)
            for b in range(NGB):
                pltpu.async_copy(
                    feat_hbm.at[pidx_v.at[pl.ds(b * GB, GB)]], gbuf_v,
                    sem).wait()
                pltpu.sync_copy(
                    gbuf_v, xperm_hbm.at[pl.ds(tid * STR + b * GB, GB)])

    return pl.kernel(
        body,
        out_type=(jax.ShapeDtypeStruct((N_PAD,), jnp.float32),
                  jax.ShapeDtypeStruct((N_PAD,), jnp.float32),
                  jax.ShapeDtypeStruct((N_PAD, DIN), jnp.float32)),
        mesh=mesh,
        scratch_types=[pltpu.VMEM((NB, 128), jnp.int32),
                       pltpu.VMEM((NB, 128), jnp.int32),
                       pltpu.VMEM((128,), jnp.float32),
                       pltpu.VMEM((STR,), jnp.int32),
                       pltpu.VMEM((GB, DIN), jnp.float32),
                       pltpu.VMEM_SHARED((N_PAD,), jnp.float32),
                       pltpu.VMEM_SHARED((N_PAD,), jnp.float32),
                       pltpu.SemaphoreType.DMA],
    )


def _make_agg_kernel(C, E_PAD):
    EPT = E_PAD // 16
    NB = EPT // 128
    NBH = NB // 2
    STR = N_PAD // 16
    CPC = C // 2
    mesh = plsc.VectorSubcoreMesh(core_axis_name="c", subcore_axis_name="s")

    def body(*refs):
        vals = refs[:C]
        src2_hbm, dst2_hbm, zeros_hbm = refs[C:C + 3]
        outs = refs[C + 3:2 * C + 3]
        sidx_v, didx_v, gbuf0_v, gbuf1_v, acc_sh, sem0, sem1 = \
            refs[2 * C + 3:]
        cid = lax.axis_index("c")
        tid = lax.axis_index("s")

        for ci in range(CPC):
            for z in range(STR // 128):
                pltpu.sync_copy(zeros_hbm,
                                acc_sh.at[pl.ds(tid * STR + z * 128, 128)])
            plsc.subcore_barrier()

            for core in range(2):
                c = core * CPC + ci

                @pl.when(cid == core)
                def _(c=c):
                    for half in range(2):
                        base = tid * NB + half * NBH
                        pltpu.sync_copy(src2_hbm.at[pl.ds(base, NBH)],
                                        sidx_v)
                        pltpu.sync_copy(dst2_hbm.at[pl.ds(base, NBH)],
                                        didx_v)
                        pltpu.async_copy(vals[c].at[sidx_v.at[0]],
                                         gbuf0_v, sem0)
                        pltpu.async_copy(vals[c].at[sidx_v.at[1]],
                                         gbuf1_v, sem1)

                        def eb(k, _):
                            for par, buf, sem in ((0, gbuf0_v, sem0),
                                                  (1, gbuf1_v, sem1)):
                                b = 2 * k + par
                                pltpu.make_async_copy(
                                    vals[c].at[sidx_v.at[b]], buf,
                                    sem).wait()
                                pltpu.sync_copy(buf,
                                                acc_sh.at[didx_v.at[b]],
                                                add=True)

                                @pl.when(b + 2 < NBH)
                                def _(buf=buf, sem=sem, b=b):
                                    pltpu.async_copy(
                                        vals[c].at[sidx_v.at[b + 2]],
                                        buf, sem)
                            return 0
                        lax.fori_loop(0, NBH // 2, eb, 0)
            plsc.subcore_barrier()

            for core in range(2):
                c = core * CPC + ci

                @pl.when(cid == core)
                def _(c=c):
                    for z in range(STR // 128):
                        pltpu.sync_copy(
                            acc_sh.at[pl.ds(tid * STR + z * 128, 128)],
                            outs[c].at[pl.ds(tid * STR + z * 128, 128)])
            plsc.subcore_barrier()

    return pl.kernel(
        body,
        out_type=tuple(jax.ShapeDtypeStruct((N_PAD, 128), jnp.float32)
                       for _ in range(C)),
        mesh=mesh,
        scratch_types=[pltpu.VMEM((NBH, 128), jnp.int32),
                       pltpu.VMEM((NBH, 128), jnp.int32),
                       pltpu.VMEM((128, 128), jnp.float32),
                       pltpu.VMEM((128, 128), jnp.float32),
                       pltpu.VMEM_SHARED((N_PAD, 128), jnp.float32),
                       pltpu.SemaphoreType.DMA,
                       pltpu.SemaphoreType.DMA],
    )


def _prep_body(feat_ref, xperm_ref, degs_ref, o0, o1, o2, o3):
    ns = lax.rsqrt(jnp.maximum(degs_ref[...], 1.0))
    xs = feat_ref[...] * ns
    xn = xperm_ref[...] * ns
    o0[...] = xs[:, :128]
    o1[...] = xs[:, 128:]
    o2[...] = xn[:, :128]
    o3[...] = xn[:, 128:]


def _l1_body(a0, a1_, a2, a3, W1_ref, b1_ref, al_ref, degs_ref, degd_ref,
             *h_refs):
    ns = lax.rsqrt(jnp.maximum(degs_ref[...], 1.0))
    nd = lax.rsqrt(jnp.maximum(degd_ref[...], 1.0))
    W1 = W1_ref[...]
    b1 = b1_ref[...]
    al = al_ref[...]
    aggP = jnp.concatenate([a0[...], a1_[...]], axis=1)
    aggN = jnp.concatenate([a2[...], a3[...]], axis=1)
    yp = jnp.dot(aggP, W1, preferred_element_type=jnp.float32) * nd + b1
    yn = jnp.dot(aggN, W1, preferred_element_type=jnp.float32) * nd + b1
    hp = jnp.where(yp >= 0, yp, al * yp) * ns
    hn = jnp.where(yn >= 0, yn, al * yn) * ns
    for k in range(4):
        h_refs[k][...] = hp[:, k * 128:(k + 1) * 128]
        h_refs[4 + k][...] = hn[:, k * 128:(k + 1) * 128]


def _l2_body(g0, g1, g2, g3, g4, g5, g6, g7, W2_ref, b2_ref, degd_ref,
             pos_ref, neg_ref, cs_ref, *, n_real):
    r = pl.program_id(0)
    nd = lax.rsqrt(jnp.maximum(degd_ref[...], 1.0))
    W2 = W2_ref[...]
    b2 = b2_ref[...]
    aggP = jnp.concatenate([g0[...], g1[...], g2[...], g3[...]], axis=1) * nd
    aggN = jnp.concatenate([g4[...], g5[...], g6[...], g7[...]], axis=1) * nd
    pos = jnp.dot(aggP, W2, preferred_element_type=jnp.float32) + b2
    neg = jnp.dot(aggN, W2, preferred_element_type=jnp.float32) + b2
    pos_ref[...] = pos
    neg_ref[...] = neg
    rows = r * RT + lax.broadcasted_iota(jnp.int32, (RT, 1), 0)
    posm = jnp.where(rows < n_real, pos, 0.0)
    cs = jnp.sum(posm, axis=0, keepdims=True)

    @pl.when(r == 0)
    def _():
        cs_ref[...] = cs

    @pl.when(r > 0)
    def _():
        cs_ref[...] += cs


def _softplus(x):
    return jnp.maximum(x, 0.0) + jnp.log(1.0 + jnp.exp(-jnp.abs(x)))


def _loss_body(cs_ref, Wd_ref, pos_ref, neg_ref, out_ref, ws_v, *, n_real):
    r = pl.program_id(0)

    @pl.when(r == 0)
    def _():
        s = jax.nn.sigmoid(cs_ref[...] * (1.0 / n_real))
        ws_v[...] = lax.dot_general(Wd_ref[...], s,
                                    (((1,), (1,)), ((), ())),
                                    preferred_element_type=jnp.float32)
        out_ref[...] = jnp.zeros((1, 1), jnp.float32)

    @pl.when(r > 0)
    def _():
        rows = (r - 1) * RT + lax.broadcasted_iota(jnp.int32, (RT, 1), 0)
        mask = rows < n_real
        ps = jnp.dot(pos_ref[...], ws_v[...],
                     preferred_element_type=jnp.float32)
        nsc = jnp.dot(neg_ref[...], ws_v[...],
                      preferred_element_type=jnp.float32)
        term = _softplus(-ps) + _softplus(nsc)
        acc = jnp.sum(jnp.where(mask, term, 0.0)) * (1.0 / n_real)
        out_ref[...] += acc.reshape(1, 1)


def kernel(feat, edge_index, W1, b1, a1, W2, b2, Wd):
    N, DIN = feat.shape
    DH = W1.shape[1]
    E = edge_index.shape[1]
    E_PAD = -(-E // 40960) * 40960
    GRID = N_PAD // RT
    f32 = jnp.float32

    perm = jax.random.permutation(jax.random.key(42), N).astype(jnp.int32)
    src = edge_index[0]
    dst = edge_index[1]
    padi = jnp.full((E_PAD - E,), N, jnp.int32)
    src_p = jnp.concatenate([src, padi])
    dst_p = jnp.concatenate([dst, padi])
    src2 = src_p.reshape(E_PAD // 128, 128)
    dst2 = dst_p.reshape(E_PAD // 128, 128)
    perm_p = jnp.concatenate([perm, jnp.zeros((N_PAD - N,), jnp.int32)])
    perm2 = perm_p.reshape(16, N_PAD // 16)
    featp = jnp.zeros((N_PAD, DIN), f32).at[:N].set(feat)
    zeros128 = jnp.zeros((128, 128), f32)
    b1r = b1.reshape(1, DH)
    a1r = a1.reshape(1, DH)
    b2r = b2.reshape(1, DH)

    zeros1 = jnp.zeros((N_PAD // 16,), f32)
    degs, degd, xperm = _make_pre_kernel(E_PAD, DIN)(
        src2, dst2, perm2, featp, zeros1)
    degs2 = degs.reshape(N_PAD, 1)
    degd2 = degd.reshape(N_PAD, 1)

    row = lambda r: (r, 0)
    fixed = lambda r: (0, 0)
    chunk_spec = pl.BlockSpec((RT, 128), row)
    b1_chunks = pl.pallas_call(
        _prep_body,
        grid=(GRID,),
        in_specs=[pl.BlockSpec((RT, DIN), row),
                  pl.BlockSpec((RT, DIN), row),
                  pl.BlockSpec((RT, 1), row)],
        out_specs=[chunk_spec] * 4,
        out_shape=[jax.ShapeDtypeStruct((N_PAD, 128), f32)] * 4,
    )(featp, xperm, degs2)

    agg1 = _make_agg_kernel(4, E_PAD)(*b1_chunks, src2, dst2, zeros128)

    h_chunks = pl.pallas_call(
        _l1_body,
        grid=(GRID,),
        in_specs=[chunk_spec] * 4 + [
            pl.BlockSpec((DIN, DH), fixed),
            pl.BlockSpec((1, DH), fixed),
            pl.BlockSpec((1, DH), fixed),
            pl.BlockSpec((RT, 1), row),
            pl.BlockSpec((RT, 1), row)],
        out_specs=[chunk_spec] * 8,
        out_shape=[jax.ShapeDtypeStruct((N_PAD, 128), f32)] * 8,
    )(*agg1, W1, b1r, a1r, degs2, degd2)

    agg2 = _make_agg_kernel(8, E_PAD)(*h_chunks, src2, dst2, zeros128)

    posA, negA, cs = pl.pallas_call(
        functools.partial(_l2_body, n_real=N),
        grid=(GRID,),
        in_specs=[chunk_spec] * 8 + [
            pl.BlockSpec((DH, DH), fixed),
            pl.BlockSpec((1, DH), fixed),
            pl.BlockSpec((RT, 1), row)],
        out_specs=[pl.BlockSpec((RT, DH), row),
                   pl.BlockSpec((RT, DH), row),
                   pl.BlockSpec((1, DH), fixed)],
        out_shape=[jax.ShapeDtypeStruct((N_PAD, DH), f32),
                   jax.ShapeDtypeStruct((N_PAD, DH), f32),
                   jax.ShapeDtypeStruct((1, DH), f32)],
    )(*agg2, W2, b2r, degd2)

    loss = pl.pallas_call(
        functools.partial(_loss_body, n_real=N),
        grid=(GRID + 1,),
        in_specs=[pl.BlockSpec((1, DH), fixed),
                  pl.BlockSpec((DH, DH), fixed),
                  pl.BlockSpec((RT, DH), lambda r: (jnp.maximum(r - 1, 0), 0)),
                  pl.BlockSpec((RT, DH), lambda r: (jnp.maximum(r - 1, 0), 0))],
        out_specs=pl.BlockSpec((1, 1), fixed),
        out_shape=jax.ShapeDtypeStruct((1, 1), f32),
        scratch_shapes=[pltpu.VMEM((DH, 1), f32)],
    )(cs, Wd, posA, negA)

    return loss[0, 0]
